# baseline (speedup 1.0000x reference)
import functools

import jax
import jax.numpy as jnp
from jax import lax
from jax.experimental import pallas as pl
from jax.experimental.pallas import tpu as pltpu
from jax.experimental.pallas import tpu_sc as plsc

_EMB = 512
_EPS = 1e-08


def _gather_centers(centers_table, label):
    B = label.shape[0]
    info = plsc.get_sparse_core_info()
    nc = info.num_cores
    nw = nc * info.num_subcores
    b_per_w = B // nw
    mesh = plsc.VectorSubcoreMesh(core_axis_name="c", subcore_axis_name="s")

    @functools.partial(
        pl.kernel,
        mesh=mesh,
        out_type=jax.ShapeDtypeStruct((B, _EMB), jnp.float32),
        scratch_types=[
            pltpu.VMEM((b_per_w,), jnp.int32),
            pltpu.VMEM((b_per_w, _EMB), jnp.float32),
            pltpu.SemaphoreType.DMA,
        ],
    )
    def gather_k(table_hbm, idx_hbm, out_hbm, idx_v, rows_v, sem):
        wid = lax.axis_index("s") * nc + lax.axis_index("c")
        base = wid * b_per_w
        pltpu.sync_copy(idx_hbm.at[pl.ds(base, b_per_w)], idx_v)
        pltpu.async_copy(table_hbm.at[idx_v], rows_v, sem).wait()
        pltpu.sync_copy(rows_v, out_hbm.at[pl.ds(base, b_per_w)])

    return gather_k(centers_table, label)


def _loss_body(x_ref, c_ref, acc_ref):
    x = x_ref[...]
    c = c_ref[...]
    bb, s, d = x.shape
    n = bb * s
    xf = x.reshape(n, d)
    ones_d = jnp.ones((d, 1), dtype=jnp.float32)
    ones_b = jnp.ones((bb, 1), dtype=jnp.float32)
    dots_all = jax.lax.dot_general(
        xf, c, (((1,), (1,)), ((), ())),
        preferred_element_type=jnp.float32)
    row = jax.lax.broadcasted_iota(jnp.int32, (n, bb), 0)
    col = jax.lax.broadcasted_iota(jnp.int32, (n, bb), 1)
    maskf = jnp.where(col == row // s, 1.0, 0.0)
    dots = jax.lax.dot(dots_all * maskf, ones_b)
    xn2 = jax.lax.dot(xf * xf, ones_d)
    cn2 = jax.lax.dot(c * c, ones_d)
    cn2_rep = jax.lax.dot(maskf, cn2)
    denom = jnp.maximum(jnp.sqrt(xn2) * jnp.sqrt(cn2_rep), _EPS)
    part = jnp.sum(dots / denom)

    @pl.when(pl.program_id(0) == 0)
    def _init():
        acc_ref[0, 0] = 0.0

    acc_ref[0, 0] += part


def kernel(x, label, centers_table):
    B, S, D = x.shape
    centers = _gather_centers(centers_table, label)
    BB = 128
    acc = pl.pallas_call(
        _loss_body,
        grid=(B // BB,),
        in_specs=[
            pl.BlockSpec((BB, S, D), lambda i: (i, 0, 0)),
            pl.BlockSpec((BB, D), lambda i: (i, 0)),
        ],
        out_specs=pl.BlockSpec(memory_space=pltpu.SMEM),
        out_shape=jax.ShapeDtypeStruct((1, 1), jnp.float32),
    )(x, centers)
    return acc[0, 0] / (B * S)


# speedup vs baseline: 1.2152x; 1.2152x over previous
import functools

import jax
import jax.numpy as jnp
from jax import lax
from jax.experimental import pallas as pl
from jax.experimental.pallas import tpu as pltpu
from jax.experimental.pallas import tpu_sc as plsc

_EMB = 512
_EPS = 1e-08


def _gather_centers(centers_table, label, off, cb):
    info = plsc.get_sparse_core_info()
    nc = info.num_cores
    nw = nc * info.num_subcores
    b_per_w = cb // nw
    mesh = plsc.VectorSubcoreMesh(core_axis_name="c", subcore_axis_name="s")

    @functools.partial(
        pl.kernel,
        mesh=mesh,
        out_type=jax.ShapeDtypeStruct((cb, _EMB), jnp.float32),
        scratch_types=[
            pltpu.VMEM((b_per_w,), jnp.int32),
            pltpu.VMEM((b_per_w, _EMB), jnp.float32),
            pltpu.SemaphoreType.DMA,
        ],
    )
    def gather_k(table_hbm, idx_hbm, out_hbm, idx_v, rows_v, sem):
        wid = lax.axis_index("s") * nc + lax.axis_index("c")
        base = wid * b_per_w
        pltpu.sync_copy(idx_hbm.at[pl.ds(off + base, b_per_w)], idx_v)
        pltpu.async_copy(table_hbm.at[idx_v], rows_v, sem).wait()
        pltpu.sync_copy(rows_v, out_hbm.at[pl.ds(base, b_per_w)])

    return gather_k(centers_table, label)


def _loss_body(x_ref, c_ref, acc_ref):
    x = x_ref[...]
    c = c_ref[...]
    dots = jnp.sum(x * c[:, None, :], axis=-1)
    xn2 = jnp.sum(x * x, axis=-1)
    cn2 = jnp.sum(c * c, axis=-1)
    q = dots * lax.rsqrt(jnp.maximum(xn2 * cn2[:, None], _EPS * _EPS))
    part = jnp.sum(q)

    @pl.when(pl.program_id(0) == 0)
    def _init():
        acc_ref[0, 0] = 0.0

    acc_ref[0, 0] += part


def _loss_chunk(x, centers, chunk, cb, bb):
    s, d = x.shape[1], x.shape[2]
    base = chunk * (cb // bb)
    return pl.pallas_call(
        _loss_body,
        grid=(cb // bb,),
        in_specs=[
            pl.BlockSpec((bb, s, d), lambda i: (base + i, 0, 0)),
            pl.BlockSpec((bb, d), lambda i: (i, 0)),
        ],
        out_specs=pl.BlockSpec(memory_space=pltpu.SMEM),
        out_shape=jax.ShapeDtypeStruct((1, 1), jnp.float32),
    )(x, centers)


def kernel(x, label, centers_table):
    B, S, D = x.shape
    NCH = 4
    CB = B // NCH
    BB = 256
    total = 0.0
    for k in range(NCH):
        centers_k = _gather_centers(centers_table, label, k * CB, CB)
        total = total + _loss_chunk(x, centers_k, k, CB, BB)[0, 0]
    return total / (B * S)


# speedup vs baseline: 1.4455x; 1.1896x over previous
import functools

import jax
import jax.numpy as jnp
from jax import lax
from jax.experimental import pallas as pl
from jax.experimental.pallas import tpu as pltpu
from jax.experimental.pallas import tpu_sc as plsc

_EMB = 512
_EPS = 1e-08


def _gather_centers(centers_table, label, off, cb):
    info = plsc.get_sparse_core_info()
    nc = info.num_cores
    nw = nc * info.num_subcores
    b_per_w = cb // nw
    mesh = plsc.VectorSubcoreMesh(core_axis_name="c", subcore_axis_name="s")

    @functools.partial(
        pl.kernel,
        mesh=mesh,
        out_type=jax.ShapeDtypeStruct((cb, _EMB), jnp.float32),
        scratch_types=[
            pltpu.VMEM((b_per_w,), jnp.int32),
            pltpu.VMEM((b_per_w, _EMB), jnp.float32),
            pltpu.SemaphoreType.DMA,
        ],
    )
    def gather_k(table_hbm, idx_hbm, out_hbm, idx_v, rows_v, sem):
        wid = lax.axis_index("s") * nc + lax.axis_index("c")
        base = wid * b_per_w
        pltpu.sync_copy(idx_hbm.at[pl.ds(off + base, b_per_w)], idx_v)
        pltpu.async_copy(table_hbm.at[idx_v], rows_v, sem).wait()
        pltpu.sync_copy(rows_v, out_hbm.at[pl.ds(base, b_per_w)])

    return gather_k(centers_table, label)


def _loss_body(x_ref, c_ref, acc_ref):
    x = x_ref[...]
    c = c_ref[...]
    dots = jnp.sum(x * c[:, None, :], axis=-1)
    xn2 = jnp.sum(x * x, axis=-1)
    cn2 = jnp.sum(c * c, axis=-1)
    q = dots * lax.rsqrt(jnp.maximum(xn2 * cn2[:, None], _EPS * _EPS))
    part = jnp.sum(q)

    @pl.when(pl.program_id(0) == 0)
    def _init():
        acc_ref[0, 0] = 0.0

    acc_ref[0, 0] += part


def _loss_chunk(x, centers, chunk, cb, bb):
    s, d = x.shape[1], x.shape[2]
    base = chunk * (cb // bb)
    return pl.pallas_call(
        _loss_body,
        grid=(cb // bb,),
        in_specs=[
            pl.BlockSpec((bb, s, d), lambda i: (base + i, 0, 0)),
            pl.BlockSpec((bb, d), lambda i: (i, 0)),
        ],
        out_specs=pl.BlockSpec(memory_space=pltpu.SMEM),
        out_shape=jax.ShapeDtypeStruct((1, 1), jnp.float32),
    )(x, centers)


def kernel(x, label, centers_table):
    B, S, D = x.shape
    BB = 512
    centers = _gather_centers(centers_table, label, 0, B)
    total = _loss_chunk(x, centers, 0, B, BB)[0, 0]
    return total / (B * S)


# speedup vs baseline: 1.4553x; 1.0067x over previous
import functools

import jax
import jax.numpy as jnp
from jax import lax
from jax.experimental import pallas as pl
from jax.experimental.pallas import tpu as pltpu
from jax.experimental.pallas import tpu_sc as plsc

_EMB = 512
_EPS = 1e-08


def _gather_centers(centers_table, label, off, cb):
    info = plsc.get_sparse_core_info()
    nc = info.num_cores
    nw = nc * info.num_subcores
    b_per_w = cb // nw
    mesh = plsc.VectorSubcoreMesh(core_axis_name="c", subcore_axis_name="s")

    @functools.partial(
        pl.kernel,
        mesh=mesh,
        out_type=jax.ShapeDtypeStruct((cb, _EMB), jnp.float32),
        scratch_types=[
            pltpu.VMEM((b_per_w,), jnp.int32),
            pltpu.VMEM((b_per_w, _EMB), jnp.float32),
            pltpu.SemaphoreType.DMA,
        ],
    )
    def gather_k(table_hbm, idx_hbm, out_hbm, idx_v, rows_v, sem):
        wid = lax.axis_index("s") * nc + lax.axis_index("c")
        base = wid * b_per_w
        pltpu.sync_copy(idx_hbm.at[pl.ds(off + base, b_per_w)], idx_v)
        pltpu.async_copy(table_hbm.at[idx_v], rows_v, sem).wait()
        pltpu.sync_copy(rows_v, out_hbm.at[pl.ds(base, b_per_w)])

    return gather_k(centers_table, label)


def _loss_body(x_ref, c_ref, acc_ref):
    x = x_ref[...]
    c = c_ref[...]
    dots = jnp.sum(x * c[:, None, :], axis=-1)
    xn2 = jnp.sum(x * x, axis=-1)
    cn2 = jnp.sum(c * c, axis=-1)
    q = dots * lax.rsqrt(jnp.maximum(xn2 * cn2[:, None], _EPS * _EPS))
    part = jnp.sum(q)

    @pl.when(pl.program_id(0) == 0)
    def _init():
        acc_ref[0, 0] = 0.0

    acc_ref[0, 0] += part


def _loss_chunk(x, centers, chunk, cb, bb):
    s, d = x.shape[1], x.shape[2]
    base = chunk * (cb // bb)
    return pl.pallas_call(
        _loss_body,
        grid=(cb // bb,),
        in_specs=[
            pl.BlockSpec((bb, s, d), lambda i: (base + i, 0, 0)),
            pl.BlockSpec((bb, d), lambda i: (i, 0)),
        ],
        out_specs=pl.BlockSpec(memory_space=pltpu.SMEM),
        out_shape=jax.ShapeDtypeStruct((1, 1), jnp.float32),
    )(x, centers)


def kernel(x, label, centers_table):
    B, S, D = x.shape
    BB = 1024
    centers = _gather_centers(centers_table, label, 0, B)
    total = _loss_chunk(x, centers, 0, B, BB)[0, 0]
    return total / (B * S)


# speedup vs baseline: 1.4568x; 1.0010x over previous
import functools

import jax
import jax.numpy as jnp
from jax import lax
from jax.experimental import pallas as pl
from jax.experimental.pallas import tpu as pltpu
from jax.experimental.pallas import tpu_sc as plsc

_EMB = 512
_EPS = 1e-08


def _gather_centers(centers_table, label, off, cb):
    info = plsc.get_sparse_core_info()
    nc = info.num_cores
    nw = nc * info.num_subcores
    b_per_w = cb // nw
    mesh = plsc.VectorSubcoreMesh(core_axis_name="c", subcore_axis_name="s")

    @functools.partial(
        pl.kernel,
        mesh=mesh,
        out_type=jax.ShapeDtypeStruct((cb, _EMB), jnp.float32),
        scratch_types=[
            pltpu.VMEM((b_per_w,), jnp.int32),
            pltpu.VMEM((b_per_w, _EMB), jnp.float32),
            pltpu.SemaphoreType.DMA,
        ],
    )
    def gather_k(table_hbm, idx_hbm, out_hbm, idx_v, rows_v, sem):
        wid = lax.axis_index("s") * nc + lax.axis_index("c")
        base = wid * b_per_w
        pltpu.sync_copy(idx_hbm.at[pl.ds(off + base, b_per_w)], idx_v)
        pltpu.async_copy(table_hbm.at[idx_v], rows_v, sem).wait()
        pltpu.sync_copy(rows_v, out_hbm.at[pl.ds(base, b_per_w)])

    return gather_k(centers_table, label)


def _loss_body(x_ref, c_ref, acc_ref):
    x = x_ref[...]
    c = c_ref[...]
    dots = jnp.sum(x * c[:, None, :], axis=-1)
    xn2 = jnp.sum(x * x, axis=-1)
    cn2 = jnp.sum(c * c, axis=-1)
    q = dots * lax.rsqrt(jnp.maximum(xn2 * cn2[:, None], _EPS * _EPS))
    part = jnp.sum(q)

    @pl.when(pl.program_id(0) == 0)
    def _init():
        acc_ref[0, 0] = 0.0

    acc_ref[0, 0] += part


def _loss_chunk(x, centers, chunk, cb, bb):
    s, d = x.shape[1], x.shape[2]
    return pl.pallas_call(
        _loss_body,
        grid=(cb // bb,),
        in_specs=[
            pl.BlockSpec((bb, s, d), lambda i: (i, 0, 0)),
            pl.BlockSpec((bb, d), lambda i: (i, 0)),
        ],
        out_specs=pl.BlockSpec(memory_space=pltpu.SMEM),
        out_shape=jax.ShapeDtypeStruct((1, 1), jnp.float32),
    )(x, centers)


def kernel(x, label, centers_table):
    B, S, D = x.shape
    BB = 1024
    centers = _gather_centers(centers_table, label, 0, B)
    total = _loss_chunk(x, centers, 0, B, BB)[0, 0]
    return total / (B * S)
